# Initial kernel scaffold; baseline (speedup 1.0000x reference)
import jax
import jax.numpy as jnp
from jax.experimental import pallas as pl


def kernel(x, edge_index, Ws, bs):
    raise NotImplementedError("write your pallas kernel here")


# speedup vs baseline: 1.7230x; 1.7230x over previous
import functools

import jax
import jax.numpy as jnp
from jax import lax
from jax.experimental import pallas as pl
from jax.experimental.pallas import tpu as pltpu
from jax.experimental.pallas import tpu_sc as plsc

N = 100000
D = 128
E = 200000
R = 3
L = 4

NT = 16
N_PAD = 100352
STRIPE = N_PAD // NT

PADR = 200704
EFLAT = 3 * PADR
SLAB_W = PADR // 32
SEGS_W = (2048, 2048, 2048, 128)

SLAB_P = PADR // NT
SEGS_P = (2048, 2048, 2048, 2048, 2048, 2048, 256)

SLAB_M = EFLAT // NT
SEGS_M = (2048,) * 18 + (768,)
NCHUNK = 7168
NPASS = 14
G = 128
LISTCAP = 2304

_f32 = jnp.float32
_i32 = jnp.int32


def _iota16():
    return lax.iota(_i32, 16)


def _rsqrt_newton(v):
    i = plsc.bitcast(v, _i32)
    i = _i32(0x5F3759DF) - lax.shift_right_logical(i, 1)
    y = plsc.bitcast(i, _f32)
    for _ in range(3):
        y = y * (1.5 - 0.5 * v * y * y)
    return y


def _prep_body(idx6_hbm, s6_hbm, parts_hbm, hist, idxb, pbuf, sf, sem):
    core = lax.axis_index("c")
    tid = lax.axis_index("s")

    czero, _ = plsc.scan_count(jnp.zeros((16,), _i32))
    cbase = jnp.full((16,), _i32(16) - czero[15], _i32)

    for a in range(3):
        arow = core * 3 + a

        def zb(i, _):
            hist[pl.ds(i * 16, 16)] = jnp.zeros((16,), _i32)
            return 0

        lax.fori_loop(0, N_PAD // 16, zb, 0)

        off = 0
        for size in SEGS_P:
            pltpu.sync_copy(idx6_hbm.at[arow, pl.ds(tid * SLAB_P + off, size)],
                            idxb.at[pl.ds(0, size)])

            def upd(j, _, off=off):
                d16 = idxb[pl.ds(j * 16, 16)]
                pos = tid * SLAB_P + off + j * 16 + _iota16()
                m = pos < E
                cnts, lastm = plsc.scan_count(d16, mask=m)
                cur = plsc.load_gather(hist, [d16])
                plsc.store_scatter(hist, [d16], cur + cnts + cbase, mask=lastm)
                return 0

            lax.fori_loop(0, size // 16, upd, 0)
            off += size

        pltpu.sync_copy(hist, parts_hbm.at[arow * NT + tid])

    plsc.subcore_barrier()

    toff = tid * STRIPE
    for a in range(3):
        arow = core * 3 + a
        pltpu.sync_copy(parts_hbm.at[arow * NT, pl.ds(toff, STRIPE)],
                        hist.at[pl.ds(0, STRIPE)])
        for p in range(1, NT):
            pltpu.sync_copy(parts_hbm.at[arow * NT + p, pl.ds(toff, STRIPE)],
                            pbuf)

            def acc(i, _):
                a16 = hist[pl.ds(i * 16, 16)]
                hist[pl.ds(i * 16, 16)] = a16 + pbuf[pl.ds(i * 16, 16)]
                return 0

            lax.fori_loop(0, STRIPE // 16, acc, 0)

        def newt(i, _):
            v = hist[pl.ds(i * 16, 16)].astype(_f32)
            sf[pl.ds(i * 16, 16)] = _rsqrt_newton(jnp.maximum(v, 1.0))
            return 0

        lax.fori_loop(0, STRIPE // 16, newt, 0)
        pltpu.sync_copy(sf, s6_hbm.at[pl.ds(arow * N_PAD + toff, STRIPE)])


def _prep(idx6):
    mesh = plsc.VectorSubcoreMesh(core_axis_name="c", subcore_axis_name="s")
    return pl.kernel(
        _prep_body,
        out_type=(jax.ShapeDtypeStruct((6 * N_PAD,), _f32),
                  jax.ShapeDtypeStruct((6 * NT, N_PAD), _i32)),
        mesh=mesh,
        scratch_types=[
            pltpu.VMEM((N_PAD,), _i32),
            pltpu.VMEM((2048,), _i32),
            pltpu.VMEM((STRIPE,), _i32),
            pltpu.VMEM((STRIPE,), _f32),
            pltpu.SemaphoreType.DMA,
        ],
        compiler_params=pltpu.CompilerParams(needs_layout_passes=False),
    )(idx6)


def _wbuild_body(dst_hbm, s6_hbm, w_hbm, sin, dstb, wob, sem):
    core = lax.axis_index("c")
    tid = lax.axis_index("s")
    wid = tid * 2 + core

    for r in range(R):
        pltpu.sync_copy(s6_hbm.at[pl.ds((3 + r) * N_PAD, N_PAD)], sin)
        off = 0
        for size in SEGS_W:
            pltpu.sync_copy(dst_hbm.at[pl.ds(r * PADR + wid * SLAB_W + off, size)],
                            dstb.at[pl.ds(0, size)])

            def gat(j, _, off=off):
                d16 = dstb[pl.ds(j * 16, 16)]
                sv = plsc.load_gather(sin, [d16])
                er = wid * SLAB_W + off + j * 16 + _iota16()
                wob[pl.ds(j * 16, 16)] = jnp.where(er < E, sv, 0.0)
                return 0

            lax.fori_loop(0, size // 16, gat, 0)
            pltpu.sync_copy(wob.at[pl.ds(0, size)],
                            w_hbm.at[pl.ds(r * PADR + wid * SLAB_W + off, size)])
            off += size


def _wbuild(dst3, s6):
    mesh = plsc.VectorSubcoreMesh(core_axis_name="c", subcore_axis_name="s")
    return pl.kernel(
        _wbuild_body,
        out_type=jax.ShapeDtypeStruct((R * PADR,), _f32),
        mesh=mesh,
        scratch_types=[
            pltpu.VMEM((N_PAD,), _f32),
            pltpu.VMEM((2048,), _i32),
            pltpu.VMEM((2048,), _f32),
            pltpu.SemaphoreType.DMA,
        ],
        compiler_params=pltpu.CompilerParams(needs_layout_passes=False),
    )(dst3.reshape(-1), s6)


def _mm_body(first, x_ref, w_ref, so_ref, b_ref, out_ref):
    xv = x_ref[...]
    if first:
        nrm = jnp.sqrt(jnp.sum(xv * xv, axis=1, keepdims=True))
        act = xv / jnp.maximum(nrm, 1e-12)
    else:
        bsum = b_ref[0] + b_ref[1] + b_ref[2]
        act = jnp.maximum(xv + bsum[None, :], 0.0)
    for r in range(R):
        hw = jnp.dot(act, w_ref[r], preferred_element_type=_f32)
        out_ref[r] = hw * so_ref[:, r][:, None]


BN = 2000


def _mm(first, x, wl, s_out, bias):
    body = functools.partial(_mm_body, first)
    return pl.pallas_call(
        body,
        grid=(N // BN,),
        in_specs=[
            pl.BlockSpec((BN, D), lambda i: (i, 0)),
            pl.BlockSpec((R, D, D), lambda i: (0, 0, 0)),
            pl.BlockSpec((BN, R), lambda i: (i, 0)),
            pl.BlockSpec((R, D), lambda i: (0, 0)),
        ],
        out_specs=pl.BlockSpec((R, BN, D), lambda i: (0, i, 0)),
        out_shape=jax.ShapeDtypeStruct((R, N, D), _f32),
    )(x, wl, s_out, bias)


def _scatter_body(final, hw_hbm, gsrc_hbm, dst_hbm, wf_hbm, bias_hbm, agg_hbm,
                  chunk_sp, dstb, gsb, wb, glist, dlist, wlist, dstage, rowbuf,
                  zbuf, biasv, bsumv, sem):
    core = lax.axis_index("c")
    tid = lax.axis_index("s")

    def zb(i, _):
        row = i // 8
        col = (i % 8) * 16
        zbuf[row, pl.ds(col, 16)] = jnp.zeros((16,), _f32)
        return 0

    lax.fori_loop(0, 64 * 8, zb, 0)

    if final:
        pltpu.sync_copy(bias_hbm, biasv)
        for jj in range(8):
            sl = pl.ds(jj * 16, 16)
            bsumv[sl] = biasv[0, sl] + biasv[1, sl] + biasv[2, sl]

    def flush(base):
        for jj in range(8):
            dstage[pl.ds(jj * 16, 16)] = dlist[pl.ds(base + jj * 16, 16)]
        pltpu.async_copy(hw_hbm.at[glist.at[pl.ds(base, G)]], rowbuf,
                         sem).wait()

        def srow(i, _):
            wv16 = wlist[pl.ds(base + i, 16)]
            wv = jnp.full((16,), wv16[0], _f32)
            for jj in range(8):
                sl = pl.ds(jj * 16, 16)
                rowbuf[i, sl] = rowbuf[i, sl] * wv
            return 0

        lax.fori_loop(0, G, srow, 0)
        pltpu.sync_copy(rowbuf, chunk_sp.at[dstage], add=True)

    def one_pass(p, _):
        base = (2 * p + core) * NCHUNK
        for k in range(NCHUNK // NT // 64):
            pltpu.sync_copy(zbuf, chunk_sp.at[pl.ds(tid * 448 + k * 64, 64)])
        plsc.subcore_barrier()

        def do_seg(off, size, cnt):
            hoff = tid * SLAB_M + off
            cp1 = pltpu.async_copy(dst_hbm.at[pl.ds(hoff, size)],
                                   dstb.at[pl.ds(0, size)], sem)
            cp2 = pltpu.async_copy(gsrc_hbm.at[pl.ds(hoff, size)],
                                   gsb.at[pl.ds(0, size)], sem)
            cp3 = pltpu.async_copy(wf_hbm.at[pl.ds(hoff, size)],
                                   wb.at[pl.ds(0, size)], sem)
            cp1.wait()
            cp2.wait()
            cp3.wait()

            def scan(j, cnt):
                sl = pl.ds(j * 16, 16)
                d16 = dstb[sl]
                m = (d16 >= base) & (d16 < base + NCHUNK)
                plsc.store_compressed(glist.at[pl.ds(cnt, 16)], gsb[sl],
                                      mask=m)
                plsc.store_compressed(dlist.at[pl.ds(cnt, 16)], d16 - base,
                                      mask=m)
                plsc.store_compressed(wlist.at[pl.ds(cnt, 16)], wb[sl],
                                      mask=m)
                return cnt + jnp.sum(m.astype(_i32))

            cnt = lax.fori_loop(0, size // 16, scan, cnt)

            nf = cnt // G

            def do_flush(k, _):
                flush(k * G)
                return 0

            lax.fori_loop(0, nf, do_flush, 0)

            @pl.when(nf > 0)
            def _():
                fb = nf * G
                for jj in range(8):
                    dsl = pl.ds(jj * 16, 16)
                    ssl = pl.ds(fb + jj * 16, 16)
                    glist[dsl] = glist[ssl]
                    dlist[dsl] = dlist[ssl]
                    wlist[dsl] = wlist[ssl]

            return cnt - nf * G

        cnt = _i32(0)
        off = 0
        for size in SEGS_M:
            cnt = do_seg(off, size, cnt)
            off += size

        @pl.when(cnt > 0)
        def _():
            for jj in range(8):
                sl = pl.ds(jj * 16, 16)
                m = (jj * 16 + _iota16()) < cnt
                glist[sl] = jnp.where(m, glist[sl], 0)
                dlist[sl] = jnp.where(m, dlist[sl], 0)
                wlist[sl] = jnp.where(m, wlist[sl], 0.0)
            flush(0)

        plsc.subcore_barrier()

        if final:
            for k in range(4):
                roff = tid * 448 + k * 112
                pltpu.sync_copy(chunk_sp.at[pl.ds(roff, 112)],
                                rowbuf.at[pl.ds(0, 112)])

                def badd(i, _):
                    for jj in range(8):
                        sl = pl.ds(jj * 16, 16)
                        rowbuf[i, sl] = rowbuf[i, sl] + bsumv[sl]
                    return 0

                lax.fori_loop(0, 112, badd, 0)
                pltpu.sync_copy(rowbuf.at[pl.ds(0, 112)],
                                agg_hbm.at[pl.ds(base + roff, 112)])
        else:
            pltpu.sync_copy(chunk_sp.at[pl.ds(tid * 448, 448)],
                            agg_hbm.at[pl.ds(base + tid * 448, 448)])
        plsc.subcore_barrier()
        return 0

    lax.fori_loop(0, NPASS // 2, one_pass, 0)


def _scatter(final, hw, gsrc, dstf, wf, bias):
    mesh = plsc.VectorSubcoreMesh(core_axis_name="c", subcore_axis_name="s")
    body = functools.partial(_scatter_body, final)
    return pl.kernel(
        body,
        out_type=jax.ShapeDtypeStruct((NPASS * NCHUNK, D), _f32),
        mesh=mesh,
        scratch_types=[
            pltpu.VMEM_SHARED((NCHUNK, D), _f32),
            pltpu.VMEM((2048,), _i32),
            pltpu.VMEM((2048,), _i32),
            pltpu.VMEM((2048,), _f32),
            pltpu.VMEM((LISTCAP,), _i32),
            pltpu.VMEM((LISTCAP,), _i32),
            pltpu.VMEM((LISTCAP,), _f32),
            pltpu.VMEM((G,), _i32),
            pltpu.VMEM((G, D), _f32),
            pltpu.VMEM((64, D), _f32),
            pltpu.VMEM((R, D), _f32),
            pltpu.VMEM((D,), _f32),
            pltpu.SemaphoreType.DMA,
        ],
        compiler_params=pltpu.CompilerParams(needs_layout_passes=False),
    )(hw, gsrc, dstf, wf, bias)


def kernel(x, edge_index, Ws, bs):
    ei = edge_index.astype(_i32)
    src3 = jnp.pad(ei[:, 0, :], ((0, 0), (0, PADR - E)))
    dst3 = jnp.pad(ei[:, 1, :], ((0, 0), (0, PADR - E)))
    gsrc = (src3 + jnp.arange(R, dtype=_i32)[:, None] * N).reshape(-1)
    dstf = dst3.reshape(-1)

    idx6 = jnp.pad(jnp.concatenate([ei[:, 0, :], ei[:, 1, :]], axis=0),
                   ((0, 0), (0, PADR - E)))

    s6, _ = _prep(idx6)
    wf = _wbuild(dst3, s6)
    s_out = s6.reshape(6, N_PAD)[0:3, :N].T

    h = x
    for l in range(L):
        hw3 = _mm(l == 0, h, Ws[l], s_out, bs[l - 1] if l > 0 else bs[0])
        hw = hw3.reshape(R * N, D)
        agg = _scatter(l == L - 1, hw, gsrc, dstf, wf, bs[L - 1])
        h = agg[:N]
    return h


# speedup vs baseline: 1.8169x; 1.0544x over previous
import functools

import jax
import jax.numpy as jnp
from jax import lax
from jax.experimental import pallas as pl
from jax.experimental.pallas import tpu as pltpu
from jax.experimental.pallas import tpu_sc as plsc

N = 100000
D = 128
E = 200000
R = 3
L = 4

NT = 16
N_PAD = 100352
STRIPE = N_PAD // NT

PADR = 200704
EFLAT = 3 * PADR
SLAB_W = PADR // 32
SEGS_W = (2048, 2048, 2048, 128)

SLAB_P = PADR // NT
SEGS_P = (2048, 2048, 2048, 2048, 2048, 2048, 256)

SLAB_M = EFLAT // NT
SEGS_M = (2048,) * 18 + (768,)
NCHUNK = 7168
NPASS = 14
TROWS = NCHUNK // NT
G = 128
LISTCAP = 2304
NSEG = len(SEGS_M)

_f32 = jnp.float32
_i32 = jnp.int32


def _iota16():
    return lax.iota(_i32, 16)


def _rsqrt_newton(v):
    i = plsc.bitcast(v, _i32)
    i = _i32(0x5F3759DF) - lax.shift_right_logical(i, 1)
    y = plsc.bitcast(i, _f32)
    for _ in range(3):
        y = y * (1.5 - 0.5 * v * y * y)
    return y


def _prep_body(idx6_hbm, s6_hbm, parts_hbm, hist, idxb, pbuf, sf, sem):
    core = lax.axis_index("c")
    tid = lax.axis_index("s")

    czero, _ = plsc.scan_count(jnp.zeros((16,), _i32))
    cbase = jnp.full((16,), _i32(16) - czero[15], _i32)

    for a in range(3):
        arow = core * 3 + a

        def zb(i, _):
            hist[pl.ds(i * 16, 16)] = jnp.zeros((16,), _i32)
            return 0

        lax.fori_loop(0, N_PAD // 16, zb, 0)

        off = 0
        for size in SEGS_P:
            pltpu.sync_copy(idx6_hbm.at[arow, pl.ds(tid * SLAB_P + off, size)],
                            idxb.at[pl.ds(0, size)])

            def upd(j, _, off=off):
                d16 = idxb[pl.ds(j * 16, 16)]
                pos = tid * SLAB_P + off + j * 16 + _iota16()
                m = pos < E
                cnts, lastm = plsc.scan_count(d16, mask=m)
                cur = plsc.load_gather(hist, [d16])
                plsc.store_scatter(hist, [d16], cur + cnts + cbase, mask=lastm)
                return 0

            lax.fori_loop(0, size // 16, upd, 0)
            off += size

        pltpu.sync_copy(hist, parts_hbm.at[arow * NT + tid])

    plsc.subcore_barrier()

    toff = tid * STRIPE
    for a in range(3):
        arow = core * 3 + a
        pltpu.sync_copy(parts_hbm.at[arow * NT, pl.ds(toff, STRIPE)],
                        hist.at[pl.ds(0, STRIPE)])
        for p in range(1, NT):
            pltpu.sync_copy(parts_hbm.at[arow * NT + p, pl.ds(toff, STRIPE)],
                            pbuf)

            def acc(i, _):
                a16 = hist[pl.ds(i * 16, 16)]
                hist[pl.ds(i * 16, 16)] = a16 + pbuf[pl.ds(i * 16, 16)]
                return 0

            lax.fori_loop(0, STRIPE // 16, acc, 0)

        def newt(i, _):
            v = hist[pl.ds(i * 16, 16)].astype(_f32)
            sf[pl.ds(i * 16, 16)] = _rsqrt_newton(jnp.maximum(v, 1.0))
            return 0

        lax.fori_loop(0, STRIPE // 16, newt, 0)
        pltpu.sync_copy(sf, s6_hbm.at[pl.ds(arow * N_PAD + toff, STRIPE)])


def _prep(idx6):
    mesh = plsc.VectorSubcoreMesh(core_axis_name="c", subcore_axis_name="s")
    return pl.kernel(
        _prep_body,
        out_type=(jax.ShapeDtypeStruct((6 * N_PAD,), _f32),
                  jax.ShapeDtypeStruct((6 * NT, N_PAD), _i32)),
        mesh=mesh,
        scratch_types=[
            pltpu.VMEM((N_PAD,), _i32),
            pltpu.VMEM((2048,), _i32),
            pltpu.VMEM((STRIPE,), _i32),
            pltpu.VMEM((STRIPE,), _f32),
            pltpu.SemaphoreType.DMA,
        ],
        compiler_params=pltpu.CompilerParams(needs_layout_passes=False),
    )(idx6)


def _wbuild_body(dst_hbm, s6_hbm, w_hbm, sin, dstb, wob, sem):
    core = lax.axis_index("c")
    tid = lax.axis_index("s")
    wid = tid * 2 + core

    for r in range(R):
        pltpu.sync_copy(s6_hbm.at[pl.ds((3 + r) * N_PAD, N_PAD)], sin)
        off = 0
        for size in SEGS_W:
            pltpu.sync_copy(dst_hbm.at[pl.ds(r * PADR + wid * SLAB_W + off, size)],
                            dstb.at[pl.ds(0, size)])

            def gat(j, _, off=off):
                d16 = dstb[pl.ds(j * 16, 16)]
                sv = plsc.load_gather(sin, [d16])
                er = wid * SLAB_W + off + j * 16 + _iota16()
                wob[pl.ds(j * 16, 16)] = jnp.where(er < E, sv, 0.0)
                return 0

            lax.fori_loop(0, size // 16, gat, 0)
            pltpu.sync_copy(wob.at[pl.ds(0, size)],
                            w_hbm.at[pl.ds(r * PADR + wid * SLAB_W + off, size)])
            off += size


def _wbuild(dst3, s6):
    mesh = plsc.VectorSubcoreMesh(core_axis_name="c", subcore_axis_name="s")
    return pl.kernel(
        _wbuild_body,
        out_type=jax.ShapeDtypeStruct((R * PADR,), _f32),
        mesh=mesh,
        scratch_types=[
            pltpu.VMEM((N_PAD,), _f32),
            pltpu.VMEM((2048,), _i32),
            pltpu.VMEM((2048,), _f32),
            pltpu.SemaphoreType.DMA,
        ],
        compiler_params=pltpu.CompilerParams(needs_layout_passes=False),
    )(dst3.reshape(-1), s6)


def _mm_body(first, x_ref, w_ref, so_ref, b_ref, out_ref):
    xv = x_ref[...]
    if first:
        nrm = jnp.sqrt(jnp.sum(xv * xv, axis=1, keepdims=True))
        act = xv / jnp.maximum(nrm, 1e-12)
    else:
        bsum = b_ref[0] + b_ref[1] + b_ref[2]
        act = jnp.maximum(xv + bsum[None, :], 0.0)
    for r in range(R):
        hw = jnp.dot(act, w_ref[r], preferred_element_type=_f32)
        out_ref[r] = hw * so_ref[:, r][:, None]


BN = 2000


def _mm(first, x, wl, s_out, bias):
    body = functools.partial(_mm_body, first)
    return pl.pallas_call(
        body,
        grid=(N // BN,),
        in_specs=[
            pl.BlockSpec((BN, D), lambda i: (i, 0)),
            pl.BlockSpec((R, D, D), lambda i: (0, 0, 0)),
            pl.BlockSpec((BN, R), lambda i: (i, 0)),
            pl.BlockSpec((R, D), lambda i: (0, 0)),
        ],
        out_specs=pl.BlockSpec((R, BN, D), lambda i: (0, i, 0)),
        out_shape=jax.ShapeDtypeStruct((R, N, D), _f32),
    )(x, wl, s_out, bias)


def _scatter_body(final, hw_hbm, gsrc_hbm, dst_hbm, wf_hbm, bias_hbm, agg_hbm,
                  chunk_sp, dstb, gsb, wb, glist, dlist, wlist, dstage,
                  rowbufA, rowbufB, zbuf, biasv, bsumv, sem, sem2):
    core = lax.axis_index("c")
    tid = lax.axis_index("s")

    def zb(i, _):
        row = i // 8
        col = (i % 8) * 16
        zbuf[row, pl.ds(col, 16)] = jnp.zeros((16,), _f32)
        return 0

    lax.fori_loop(0, 112 * 8, zb, 0)

    if final:
        pltpu.sync_copy(bias_hbm, biasv)
        for jj in range(8):
            sl = pl.ds(jj * 16, 16)
            bsumv[sl] = biasv[0, sl] + biasv[1, sl] + biasv[2, sl]

    def fire(base, rbuf):
        return pltpu.async_copy(hw_hbm.at[glist.at[pl.ds(base, G)]], rbuf,
                                sem2)

    def process(base, cp, rbuf):
        for jj in range(8):
            dstage[pl.ds(jj * 16, 16)] = dlist[pl.ds(base + jj * 16, 16)]
        cp.wait()

        def srow(i, _):
            wv16 = wlist[pl.ds(base + i, 16)]
            wv = jnp.full((16,), wv16[0], _f32)
            for jj in range(8):
                sl = pl.ds(jj * 16, 16)
                rbuf[i, sl] = rbuf[i, sl] * wv
            return 0

        lax.fori_loop(0, G, srow, 0)
        pltpu.sync_copy(rbuf, chunk_sp.at[dstage], add=True)

    def do_flushes(nf):
        def pair(kk, _):
            b0 = (2 * kk) * G
            b1 = b0 + G
            cpA = fire(b0, rowbufA)
            cpB = fire(b1, rowbufB)
            process(b0, cpA, rowbufA)
            process(b1, cpB, rowbufB)
            return 0

        lax.fori_loop(0, nf // 2, pair, 0)

        @pl.when(nf % 2 == 1)
        def _():
            b = (nf - 1) * G
            process(b, fire(b, rowbufA), rowbufA)

    def issue_seg(si):
        off = si * 2048
        size = SEGS_M[si]
        par = (si % 2) * 2048
        hoff = tid * SLAB_M + off
        c1 = pltpu.async_copy(dst_hbm.at[pl.ds(hoff, size)],
                              dstb.at[pl.ds(par, size)], sem)
        c2 = pltpu.async_copy(gsrc_hbm.at[pl.ds(hoff, size)],
                              gsb.at[pl.ds(par, size)], sem)
        c3 = pltpu.async_copy(wf_hbm.at[pl.ds(hoff, size)],
                              wb.at[pl.ds(par, size)], sem)
        return (c1, c2, c3)

    def one_pass(p, _):
        base = (2 * p + core) * NCHUNK
        for k in range(TROWS // 112):
            pltpu.sync_copy(zbuf, chunk_sp.at[pl.ds(tid * TROWS + k * 112,
                                                    112)])
        plsc.subcore_barrier()

        cnt = _i32(0)
        cps = issue_seg(0)
        for si in range(NSEG):
            size = SEGS_M[si]
            par = (si % 2) * 2048
            for c in cps:
                c.wait()
            if si + 1 < NSEG:
                cps = issue_seg(si + 1)

            def scan(j, cnt, par=par):
                sl = pl.ds(par + j * 16, 16)
                d16 = dstb[sl]
                m = (d16 >= base) & (d16 < base + NCHUNK)
                plsc.store_compressed(glist.at[pl.ds(cnt, 16)], gsb[sl],
                                      mask=m)
                plsc.store_compressed(dlist.at[pl.ds(cnt, 16)], d16 - base,
                                      mask=m)
                plsc.store_compressed(wlist.at[pl.ds(cnt, 16)], wb[sl],
                                      mask=m)
                return cnt + jnp.sum(m.astype(_i32))

            cnt = lax.fori_loop(0, size // 16, scan, cnt)

            nf = cnt // G
            do_flushes(nf)

            @pl.when(nf > 0)
            def _():
                fb = nf * G
                for jj in range(8):
                    dsl = pl.ds(jj * 16, 16)
                    ssl = pl.ds(fb + jj * 16, 16)
                    glist[dsl] = glist[ssl]
                    dlist[dsl] = dlist[ssl]
                    wlist[dsl] = wlist[ssl]

            cnt = cnt - nf * G

        @pl.when(cnt > 0)
        def _():
            for jj in range(8):
                sl = pl.ds(jj * 16, 16)
                m = (jj * 16 + _iota16()) < cnt
                glist[sl] = jnp.where(m, glist[sl], 0)
                dlist[sl] = jnp.where(m, dlist[sl], 0)
                wlist[sl] = jnp.where(m, wlist[sl], 0.0)
            process(0, fire(0, rowbufA), rowbufA)

        plsc.subcore_barrier()

        if final:
            for k in range(TROWS // 112):
                roff = tid * TROWS + k * 112
                pltpu.sync_copy(chunk_sp.at[pl.ds(roff, 112)],
                                rowbufA.at[pl.ds(0, 112)])

                def badd(i, _):
                    for jj in range(8):
                        sl = pl.ds(jj * 16, 16)
                        rowbufA[i, sl] = rowbufA[i, sl] + bsumv[sl]
                    return 0

                lax.fori_loop(0, 112, badd, 0)
                pltpu.sync_copy(rowbufA.at[pl.ds(0, 112)],
                                agg_hbm.at[pl.ds(base + roff, 112)])
        else:
            pltpu.sync_copy(chunk_sp.at[pl.ds(tid * TROWS, TROWS)],
                            agg_hbm.at[pl.ds(base + tid * TROWS, TROWS)])
        plsc.subcore_barrier()
        return 0

    lax.fori_loop(0, NPASS // 2, one_pass, 0)


def _scatter(final, hw, gsrc, dstf, wf, bias):
    mesh = plsc.VectorSubcoreMesh(core_axis_name="c", subcore_axis_name="s")
    body = functools.partial(_scatter_body, final)
    return pl.kernel(
        body,
        out_type=jax.ShapeDtypeStruct((NPASS * NCHUNK, D), _f32),
        mesh=mesh,
        scratch_types=[
            pltpu.VMEM_SHARED((NCHUNK, D), _f32),
            pltpu.VMEM((4096,), _i32),
            pltpu.VMEM((4096,), _i32),
            pltpu.VMEM((4096,), _f32),
            pltpu.VMEM((LISTCAP,), _i32),
            pltpu.VMEM((LISTCAP,), _i32),
            pltpu.VMEM((LISTCAP,), _f32),
            pltpu.VMEM((G,), _i32),
            pltpu.VMEM((G, D), _f32),
            pltpu.VMEM((G, D), _f32),
            pltpu.VMEM((112, D), _f32),
            pltpu.VMEM((R, D), _f32),
            pltpu.VMEM((D,), _f32),
            pltpu.SemaphoreType.DMA,
            pltpu.SemaphoreType.DMA,
        ],
        compiler_params=pltpu.CompilerParams(needs_layout_passes=False),
    )(hw, gsrc, dstf, wf, bias)


def kernel(x, edge_index, Ws, bs):
    ei = edge_index.astype(_i32)
    src3 = jnp.pad(ei[:, 0, :], ((0, 0), (0, PADR - E)))
    dst3 = jnp.pad(ei[:, 1, :], ((0, 0), (0, PADR - E)))
    gsrc = (src3 + jnp.arange(R, dtype=_i32)[:, None] * N).reshape(-1)
    dstf = dst3.reshape(-1)

    idx6 = jnp.pad(jnp.concatenate([ei[:, 0, :], ei[:, 1, :]], axis=0),
                   ((0, 0), (0, PADR - E)))

    s6, _ = _prep(idx6)
    wf = _wbuild(dst3, s6)
    s_out = s6.reshape(6, N_PAD)[0:3, :N].T

    h = x
    for l in range(L):
        hw3 = _mm(l == 0, h, Ws[l], s_out, bs[l - 1] if l > 0 else bs[0])
        hw = hw3.reshape(R * N, D)
        agg = _scatter(l == L - 1, hw, gsrc, dstf, wf, bs[L - 1])
        h = agg[:N]
    return h


# speedup vs baseline: 1.8243x; 1.0041x over previous
import functools

import jax
import jax.numpy as jnp
from jax import lax
from jax.experimental import pallas as pl
from jax.experimental.pallas import tpu as pltpu
from jax.experimental.pallas import tpu_sc as plsc

N = 100000
D = 128
E = 200000
R = 3
L = 4

NT = 16
N_PAD = 100352
STRIPE = N_PAD // NT

PADR = 200704
EFLAT = 3 * PADR
SLAB_W = PADR // 32
SEGS_W = (2048, 2048, 2048, 128)

SLAB_P = PADR // NT
SEGS_P = (2048, 2048, 2048, 2048, 2048, 2048, 256)

SLAB_M = EFLAT // NT
SEGS_M = (2048,) * 18 + (768,)
NCHUNK = 7168
NPASS = 14
TROWS = NCHUNK // NT
G = 128
LISTCAP = 2304
NSEG = len(SEGS_M)

_f32 = jnp.float32
_i32 = jnp.int32


def _iota16():
    return lax.iota(_i32, 16)


def _rsqrt_newton(v):
    i = plsc.bitcast(v, _i32)
    i = _i32(0x5F3759DF) - lax.shift_right_logical(i, 1)
    y = plsc.bitcast(i, _f32)
    for _ in range(3):
        y = y * (1.5 - 0.5 * v * y * y)
    return y


def _prep_body(idx6_hbm, s6_hbm, parts_hbm, hist, idxb, pbuf, sf, sem):
    core = lax.axis_index("c")
    tid = lax.axis_index("s")

    czero, _ = plsc.scan_count(jnp.zeros((16,), _i32))
    cbase = jnp.full((16,), _i32(16) - czero[15], _i32)

    for a in range(3):
        arow = core * 3 + a

        def zb(i, _):
            hist[pl.ds(i * 16, 16)] = jnp.zeros((16,), _i32)
            return 0

        lax.fori_loop(0, N_PAD // 16, zb, 0)

        off = 0
        for size in SEGS_P:
            pltpu.sync_copy(idx6_hbm.at[arow, pl.ds(tid * SLAB_P + off, size)],
                            idxb.at[pl.ds(0, size)])

            def upd(j, _, off=off):
                d16 = idxb[pl.ds(j * 16, 16)]
                pos = tid * SLAB_P + off + j * 16 + _iota16()
                m = pos < E
                cnts, lastm = plsc.scan_count(d16, mask=m)
                cur = plsc.load_gather(hist, [d16])
                plsc.store_scatter(hist, [d16], cur + cnts + cbase, mask=lastm)
                return 0

            lax.fori_loop(0, size // 16, upd, 0)
            off += size

        pltpu.sync_copy(hist, parts_hbm.at[arow * NT + tid])

    plsc.subcore_barrier()

    toff = tid * STRIPE
    for a in range(3):
        arow = core * 3 + a
        pltpu.sync_copy(parts_hbm.at[arow * NT, pl.ds(toff, STRIPE)],
                        hist.at[pl.ds(0, STRIPE)])
        for p in range(1, NT):
            pltpu.sync_copy(parts_hbm.at[arow * NT + p, pl.ds(toff, STRIPE)],
                            pbuf)

            def acc(i, _):
                a16 = hist[pl.ds(i * 16, 16)]
                hist[pl.ds(i * 16, 16)] = a16 + pbuf[pl.ds(i * 16, 16)]
                return 0

            lax.fori_loop(0, STRIPE // 16, acc, 0)

        def newt(i, _):
            v = hist[pl.ds(i * 16, 16)].astype(_f32)
            sf[pl.ds(i * 16, 16)] = _rsqrt_newton(jnp.maximum(v, 1.0))
            return 0

        lax.fori_loop(0, STRIPE // 16, newt, 0)
        pltpu.sync_copy(sf, s6_hbm.at[pl.ds(arow * N_PAD + toff, STRIPE)])


def _prep(idx6):
    mesh = plsc.VectorSubcoreMesh(core_axis_name="c", subcore_axis_name="s")
    return pl.kernel(
        _prep_body,
        out_type=(jax.ShapeDtypeStruct((6 * N_PAD,), _f32),
                  jax.ShapeDtypeStruct((6 * NT, N_PAD), _i32)),
        mesh=mesh,
        scratch_types=[
            pltpu.VMEM((N_PAD,), _i32),
            pltpu.VMEM((2048,), _i32),
            pltpu.VMEM((STRIPE,), _i32),
            pltpu.VMEM((STRIPE,), _f32),
            pltpu.SemaphoreType.DMA,
        ],
        compiler_params=pltpu.CompilerParams(needs_layout_passes=False),
    )(idx6)


def _wbuild_body(dst_hbm, s6_hbm, w_hbm, sin, dstb, wob, sem):
    core = lax.axis_index("c")
    tid = lax.axis_index("s")
    wid = tid * 2 + core

    for r in range(R):
        pltpu.sync_copy(s6_hbm.at[pl.ds((3 + r) * N_PAD, N_PAD)], sin)
        off = 0
        for size in SEGS_W:
            pltpu.sync_copy(dst_hbm.at[pl.ds(r * PADR + wid * SLAB_W + off, size)],
                            dstb.at[pl.ds(0, size)])

            def gat(j, _, off=off):
                d16 = dstb[pl.ds(j * 16, 16)]
                sv = plsc.load_gather(sin, [d16])
                er = wid * SLAB_W + off + j * 16 + _iota16()
                wob[pl.ds(j * 16, 16)] = jnp.where(er < E, sv, 0.0)
                return 0

            lax.fori_loop(0, size // 16, gat, 0)
            pltpu.sync_copy(wob.at[pl.ds(0, size)],
                            w_hbm.at[pl.ds(r * PADR + wid * SLAB_W + off, size)])
            off += size


def _wbuild(dst3, s6):
    mesh = plsc.VectorSubcoreMesh(core_axis_name="c", subcore_axis_name="s")
    return pl.kernel(
        _wbuild_body,
        out_type=jax.ShapeDtypeStruct((R * PADR,), _f32),
        mesh=mesh,
        scratch_types=[
            pltpu.VMEM((N_PAD,), _f32),
            pltpu.VMEM((2048,), _i32),
            pltpu.VMEM((2048,), _f32),
            pltpu.SemaphoreType.DMA,
        ],
        compiler_params=pltpu.CompilerParams(needs_layout_passes=False),
    )(dst3.reshape(-1), s6)


def _mm_body(first, x_ref, w_ref, so_ref, b_ref, out_ref):
    xv = x_ref[...]
    if first:
        nrm = jnp.sqrt(jnp.sum(xv * xv, axis=1, keepdims=True))
        act = xv / jnp.maximum(nrm, 1e-12)
    else:
        bsum = b_ref[0] + b_ref[1] + b_ref[2]
        act = jnp.maximum(xv + bsum[None, :], 0.0)
    actb = act.astype(jnp.bfloat16)
    for r in range(R):
        hw = jnp.dot(actb, w_ref[r].astype(jnp.bfloat16),
                     preferred_element_type=_f32)
        out_ref[r] = hw * so_ref[:, r][:, None]


BN = 2000


def _mm(first, x, wl, s_out, bias):
    body = functools.partial(_mm_body, first)
    return pl.pallas_call(
        body,
        grid=(N // BN,),
        in_specs=[
            pl.BlockSpec((BN, D), lambda i: (i, 0)),
            pl.BlockSpec((R, D, D), lambda i: (0, 0, 0)),
            pl.BlockSpec((BN, R), lambda i: (i, 0)),
            pl.BlockSpec((R, D), lambda i: (0, 0)),
        ],
        out_specs=pl.BlockSpec((R, BN, D), lambda i: (0, i, 0)),
        out_shape=jax.ShapeDtypeStruct((R, N, D), _f32),
    )(x, wl, s_out, bias)


def _scatter_body(final, hw_hbm, gsrc_hbm, dst_hbm, wf_hbm, bias_hbm, agg_hbm,
                  chunk_sp, dstb, gsb, wb, glist, dlist, wlist, dstage,
                  dstageB, rowbufA, rowbufB, zbuf, biasv, bsumv, sem, sem2,
                  sem3):
    core = lax.axis_index("c")
    tid = lax.axis_index("s")

    def zb(i, _):
        row = i // 8
        col = (i % 8) * 16
        zbuf[row, pl.ds(col, 16)] = jnp.zeros((16,), _f32)
        return 0

    lax.fori_loop(0, 112 * 8, zb, 0)

    if final:
        pltpu.sync_copy(bias_hbm, biasv)
        for jj in range(8):
            sl = pl.ds(jj * 16, 16)
            bsumv[sl] = biasv[0, sl] + biasv[1, sl] + biasv[2, sl]

    def fire(base, rbuf):
        return pltpu.async_copy(hw_hbm.at[glist.at[pl.ds(base, G)]], rbuf,
                                sem2)

    def scale_rows(base, rbuf):
        def srow(i, _):
            wv16 = wlist[pl.ds(base + i, 16)]
            wv = jnp.full((16,), wv16[0], _f32)
            for jj in range(8):
                sl = pl.ds(jj * 16, 16)
                rbuf[i, sl] = rbuf[i, sl] * wv
            return 0

        lax.fori_loop(0, G, srow, 0)

    def stage_idx(base, dst_stage):
        for jj in range(8):
            dst_stage[pl.ds(jj * 16, 16)] = dlist[pl.ds(base + jj * 16, 16)]

    def process(base, cp, rbuf):
        stage_idx(base, dstage)
        cp.wait()
        scale_rows(base, rbuf)
        pltpu.sync_copy(rbuf, chunk_sp.at[dstage], add=True)

    def do_flushes(nf):
        def pair(kk, _):
            b0 = (2 * kk) * G
            b1 = b0 + G
            cpA = fire(b0, rowbufA)
            cpB = fire(b1, rowbufB)
            stage_idx(b0, dstage)
            cpA.wait()
            scale_rows(b0, rowbufA)
            scA = pltpu.async_copy(rowbufA, chunk_sp.at[dstage], sem3,
                                   add=True)
            stage_idx(b1, dstageB)
            cpB.wait()
            scale_rows(b1, rowbufB)
            scA.wait()
            pltpu.sync_copy(rowbufB, chunk_sp.at[dstageB], add=True)
            return 0

        lax.fori_loop(0, nf // 2, pair, 0)

        @pl.when(nf % 2 == 1)
        def _():
            b = (nf - 1) * G
            process(b, fire(b, rowbufA), rowbufA)

    def issue_seg(si):
        off = si * 2048
        size = SEGS_M[si]
        par = (si % 2) * 2048
        hoff = tid * SLAB_M + off
        c1 = pltpu.async_copy(dst_hbm.at[pl.ds(hoff, size)],
                              dstb.at[pl.ds(par, size)], sem)
        c2 = pltpu.async_copy(gsrc_hbm.at[pl.ds(hoff, size)],
                              gsb.at[pl.ds(par, size)], sem)
        c3 = pltpu.async_copy(wf_hbm.at[pl.ds(hoff, size)],
                              wb.at[pl.ds(par, size)], sem)
        return (c1, c2, c3)

    def one_pass(p, _):
        base = (2 * p + core) * NCHUNK
        for k in range(TROWS // 112):
            pltpu.sync_copy(zbuf, chunk_sp.at[pl.ds(tid * TROWS + k * 112,
                                                    112)])
        plsc.subcore_barrier()

        cnt = _i32(0)
        cps = issue_seg(0)
        for si in range(NSEG):
            size = SEGS_M[si]
            par = (si % 2) * 2048
            for c in cps:
                c.wait()
            if si + 1 < NSEG:
                cps = issue_seg(si + 1)

            def scan(j, cnt, par=par):
                sl = pl.ds(par + j * 16, 16)
                d16 = dstb[sl]
                m = (d16 >= base) & (d16 < base + NCHUNK)
                plsc.store_compressed(glist.at[pl.ds(cnt, 16)], gsb[sl],
                                      mask=m)
                plsc.store_compressed(dlist.at[pl.ds(cnt, 16)], d16 - base,
                                      mask=m)
                plsc.store_compressed(wlist.at[pl.ds(cnt, 16)], wb[sl],
                                      mask=m)
                return cnt + jnp.sum(m.astype(_i32))

            cnt = lax.fori_loop(0, size // 16, scan, cnt)

            nf = cnt // G
            do_flushes(nf)

            @pl.when(nf > 0)
            def _():
                fb = nf * G
                for jj in range(8):
                    dsl = pl.ds(jj * 16, 16)
                    ssl = pl.ds(fb + jj * 16, 16)
                    glist[dsl] = glist[ssl]
                    dlist[dsl] = dlist[ssl]
                    wlist[dsl] = wlist[ssl]

            cnt = cnt - nf * G

        @pl.when(cnt > 0)
        def _():
            for jj in range(8):
                sl = pl.ds(jj * 16, 16)
                m = (jj * 16 + _iota16()) < cnt
                glist[sl] = jnp.where(m, glist[sl], 0)
                dlist[sl] = jnp.where(m, dlist[sl], 0)
                wlist[sl] = jnp.where(m, wlist[sl], 0.0)
            process(0, fire(0, rowbufA), rowbufA)

        plsc.subcore_barrier()

        if final:
            for k in range(TROWS // 112):
                roff = tid * TROWS + k * 112
                pltpu.sync_copy(chunk_sp.at[pl.ds(roff, 112)],
                                rowbufA.at[pl.ds(0, 112)])

                def badd(i, _):
                    for jj in range(8):
                        sl = pl.ds(jj * 16, 16)
                        rowbufA[i, sl] = rowbufA[i, sl] + bsumv[sl]
                    return 0

                lax.fori_loop(0, 112, badd, 0)
                pltpu.sync_copy(rowbufA.at[pl.ds(0, 112)],
                                agg_hbm.at[pl.ds(base + roff, 112)])
        else:
            pltpu.sync_copy(chunk_sp.at[pl.ds(tid * TROWS, TROWS)],
                            agg_hbm.at[pl.ds(base + tid * TROWS, TROWS)])
        plsc.subcore_barrier()
        return 0

    lax.fori_loop(0, NPASS // 2, one_pass, 0)


def _scatter(final, hw, gsrc, dstf, wf, bias):
    mesh = plsc.VectorSubcoreMesh(core_axis_name="c", subcore_axis_name="s")
    body = functools.partial(_scatter_body, final)
    return pl.kernel(
        body,
        out_type=jax.ShapeDtypeStruct((NPASS * NCHUNK, D), _f32),
        mesh=mesh,
        scratch_types=[
            pltpu.VMEM_SHARED((NCHUNK, D), _f32),
            pltpu.VMEM((4096,), _i32),
            pltpu.VMEM((4096,), _i32),
            pltpu.VMEM((4096,), _f32),
            pltpu.VMEM((LISTCAP,), _i32),
            pltpu.VMEM((LISTCAP,), _i32),
            pltpu.VMEM((LISTCAP,), _f32),
            pltpu.VMEM((G,), _i32),
            pltpu.VMEM((G,), _i32),
            pltpu.VMEM((G, D), _f32),
            pltpu.VMEM((G, D), _f32),
            pltpu.VMEM((112, D), _f32),
            pltpu.VMEM((R, D), _f32),
            pltpu.VMEM((D,), _f32),
            pltpu.SemaphoreType.DMA,
            pltpu.SemaphoreType.DMA,
            pltpu.SemaphoreType.DMA,
        ],
        compiler_params=pltpu.CompilerParams(needs_layout_passes=False),
    )(hw, gsrc, dstf, wf, bias)


def kernel(x, edge_index, Ws, bs):
    ei = edge_index.astype(_i32)
    src3 = jnp.pad(ei[:, 0, :], ((0, 0), (0, PADR - E)))
    dst3 = jnp.pad(ei[:, 1, :], ((0, 0), (0, PADR - E)))
    gsrc = (src3 + jnp.arange(R, dtype=_i32)[:, None] * N).reshape(-1)
    dstf = dst3.reshape(-1)

    idx6 = jnp.pad(jnp.concatenate([ei[:, 0, :], ei[:, 1, :]], axis=0),
                   ((0, 0), (0, PADR - E)))

    s6, _ = _prep(idx6)
    wf = _wbuild(dst3, s6)
    s_out = s6.reshape(6, N_PAD)[0:3, :N].T

    h = x
    for l in range(L):
        hw3 = _mm(l == 0, h, Ws[l], s_out, bs[l - 1] if l > 0 else bs[0])
        hw = hw3.reshape(R * N, D)
        agg = _scatter(l == L - 1, hw, gsrc, dstf, wf, bs[L - 1])
        h = agg[:N]
    return h


# speedup vs baseline: 1.8273x; 1.0016x over previous
import functools

import jax
import jax.numpy as jnp
from jax import lax
from jax.experimental import pallas as pl
from jax.experimental.pallas import tpu as pltpu
from jax.experimental.pallas import tpu_sc as plsc

N = 100000
D = 128
E = 200000
R = 3
L = 4

NT = 16
N_PAD = 100352
STRIPE = N_PAD // NT

PADR = 200704
EFLAT = 3 * PADR
SLAB_W = PADR // 32
SEGS_W = (2048, 2048, 2048, 128)

SLAB_P = PADR // NT
SEGS_P = (2048, 2048, 2048, 2048, 2048, 2048, 256)

SLAB_M = EFLAT // NT
SEGS_M = (2048,) * 18 + (768,)
NCHUNK = 7168
NPASS = 14
TROWS = NCHUNK // NT
G = 128
LISTCAP = 2304
NSEG = len(SEGS_M)

_f32 = jnp.float32
_i32 = jnp.int32


def _iota16():
    return lax.iota(_i32, 16)


def _rsqrt_newton(v):
    i = plsc.bitcast(v, _i32)
    i = _i32(0x5F3759DF) - lax.shift_right_logical(i, 1)
    y = plsc.bitcast(i, _f32)
    for _ in range(3):
        y = y * (1.5 - 0.5 * v * y * y)
    return y


def _prep_body(idx6_hbm, s6_hbm, parts_hbm, hist, idxb, pbuf, sf, sem):
    core = lax.axis_index("c")
    tid = lax.axis_index("s")

    czero, _ = plsc.scan_count(jnp.zeros((16,), _i32))
    cbase = jnp.full((16,), _i32(16) - czero[15], _i32)

    for a in range(3):
        arow = core * 3 + a

        def zb(i, _):
            hist[pl.ds(i * 16, 16)] = jnp.zeros((16,), _i32)
            return 0

        lax.fori_loop(0, N_PAD // 16, zb, 0)

        off = 0
        for size in SEGS_P:
            pltpu.sync_copy(idx6_hbm.at[arow, pl.ds(tid * SLAB_P + off, size)],
                            idxb.at[pl.ds(0, size)])

            def upd(j, _, off=off):
                d16 = idxb[pl.ds(j * 16, 16)]
                pos = tid * SLAB_P + off + j * 16 + _iota16()
                m = pos < E
                cnts, lastm = plsc.scan_count(d16, mask=m)
                cur = plsc.load_gather(hist, [d16])
                plsc.store_scatter(hist, [d16], cur + cnts + cbase, mask=lastm)
                return 0

            lax.fori_loop(0, size // 16, upd, 0)
            off += size

        pltpu.sync_copy(hist, parts_hbm.at[arow * NT + tid])

    plsc.subcore_barrier()

    toff = tid * STRIPE
    for a in range(3):
        arow = core * 3 + a
        pltpu.sync_copy(parts_hbm.at[arow * NT, pl.ds(toff, STRIPE)],
                        hist.at[pl.ds(0, STRIPE)])
        for p in range(1, NT):
            pltpu.sync_copy(parts_hbm.at[arow * NT + p, pl.ds(toff, STRIPE)],
                            pbuf)

            def acc(i, _):
                a16 = hist[pl.ds(i * 16, 16)]
                hist[pl.ds(i * 16, 16)] = a16 + pbuf[pl.ds(i * 16, 16)]
                return 0

            lax.fori_loop(0, STRIPE // 16, acc, 0)

        def newt(i, _):
            v = hist[pl.ds(i * 16, 16)].astype(_f32)
            sf[pl.ds(i * 16, 16)] = _rsqrt_newton(jnp.maximum(v, 1.0))
            return 0

        lax.fori_loop(0, STRIPE // 16, newt, 0)
        pltpu.sync_copy(sf, s6_hbm.at[pl.ds(arow * N_PAD + toff, STRIPE)])


def _prep(idx6):
    mesh = plsc.VectorSubcoreMesh(core_axis_name="c", subcore_axis_name="s")
    return pl.kernel(
        _prep_body,
        out_type=(jax.ShapeDtypeStruct((6 * N_PAD,), _f32),
                  jax.ShapeDtypeStruct((6 * NT, N_PAD), _i32)),
        mesh=mesh,
        scratch_types=[
            pltpu.VMEM((N_PAD,), _i32),
            pltpu.VMEM((2048,), _i32),
            pltpu.VMEM((STRIPE,), _i32),
            pltpu.VMEM((STRIPE,), _f32),
            pltpu.SemaphoreType.DMA,
        ],
        compiler_params=pltpu.CompilerParams(needs_layout_passes=False),
    )(idx6)


def _wbuild_body(dst_hbm, s6_hbm, w_hbm, sin, dstb, wob, sem):
    core = lax.axis_index("c")
    tid = lax.axis_index("s")
    wid = tid * 2 + core

    for r in range(R):
        pltpu.sync_copy(s6_hbm.at[pl.ds((3 + r) * N_PAD, N_PAD)], sin)
        off = 0
        for size in SEGS_W:
            pltpu.sync_copy(dst_hbm.at[pl.ds(r * PADR + wid * SLAB_W + off, size)],
                            dstb.at[pl.ds(0, size)])

            def gat(j, _, off=off):
                d16 = dstb[pl.ds(j * 16, 16)]
                sv = plsc.load_gather(sin, [d16])
                er = wid * SLAB_W + off + j * 16 + _iota16()
                wob[pl.ds(j * 16, 16)] = jnp.where(er < E, sv, 0.0)
                return 0

            lax.fori_loop(0, size // 16, gat, 0)
            pltpu.sync_copy(wob.at[pl.ds(0, size)],
                            w_hbm.at[pl.ds(r * PADR + wid * SLAB_W + off, size)])
            off += size


def _wbuild(dst3, s6):
    mesh = plsc.VectorSubcoreMesh(core_axis_name="c", subcore_axis_name="s")
    return pl.kernel(
        _wbuild_body,
        out_type=jax.ShapeDtypeStruct((R * PADR,), _f32),
        mesh=mesh,
        scratch_types=[
            pltpu.VMEM((N_PAD,), _f32),
            pltpu.VMEM((2048,), _i32),
            pltpu.VMEM((2048,), _f32),
            pltpu.SemaphoreType.DMA,
        ],
        compiler_params=pltpu.CompilerParams(needs_layout_passes=False),
    )(dst3.reshape(-1), s6)


def _mm_body(first, x_ref, w_ref, so_ref, b_ref, out_ref):
    xv = x_ref[...]
    if first:
        nrm = jnp.sqrt(jnp.sum(xv * xv, axis=1, keepdims=True))
        act = xv / jnp.maximum(nrm, 1e-12)
    else:
        bsum = b_ref[0] + b_ref[1] + b_ref[2]
        act = jnp.maximum(xv + bsum[None, :], 0.0)
    actb = act.astype(jnp.bfloat16)
    for r in range(R):
        hw = jnp.dot(actb, w_ref[r].astype(jnp.bfloat16),
                     preferred_element_type=_f32)
        out_ref[r] = hw * so_ref[:, r][:, None]


BN = 2000


def _mm(first, x, wl, s_out, bias):
    body = functools.partial(_mm_body, first)
    return pl.pallas_call(
        body,
        grid=(N // BN,),
        in_specs=[
            pl.BlockSpec((BN, D), lambda i: (i, 0)),
            pl.BlockSpec((R, D, D), lambda i: (0, 0, 0)),
            pl.BlockSpec((BN, R), lambda i: (i, 0)),
            pl.BlockSpec((R, D), lambda i: (0, 0)),
        ],
        out_specs=pl.BlockSpec((R, BN, D), lambda i: (0, i, 0)),
        out_shape=jax.ShapeDtypeStruct((R, N, D), _f32),
    )(x, wl, s_out, bias)


def _scatter_body(final, hw_hbm, gsrc_hbm, dst_hbm, wf_hbm, bias_hbm, agg_hbm,
                  chunk_sp, dstb, gsb, wb, glist, dlist, wlist, dstage,
                  dstageB, rowbufA, rowbufB, zbuf, biasv, bsumv, sem, sem2,
                  sem3):
    core = lax.axis_index("c")
    tid = lax.axis_index("s")

    def zb(i, _):
        row = i // 8
        col = (i % 8) * 16
        zbuf[row, pl.ds(col, 16)] = jnp.zeros((16,), _f32)
        return 0

    lax.fori_loop(0, 112 * 8, zb, 0)

    if final:
        pltpu.sync_copy(bias_hbm, biasv)
        for jj in range(8):
            sl = pl.ds(jj * 16, 16)
            bsumv[sl] = biasv[0, sl] + biasv[1, sl] + biasv[2, sl]

    def fire(base, rbuf):
        return pltpu.async_copy(hw_hbm.at[glist.at[pl.ds(base, G)]], rbuf,
                                sem2)

    def scale_rows(base, rbuf):
        def srow(i, _):
            wv16 = wlist[pl.ds(base + i, 16)]
            wv = jnp.full((16,), wv16[0], _f32)
            for jj in range(8):
                sl = pl.ds(jj * 16, 16)
                rbuf[i, sl] = rbuf[i, sl] * wv
            return 0

        lax.fori_loop(0, G, srow, 0)

    def stage_idx(base, dst_stage):
        for jj in range(8):
            dst_stage[pl.ds(jj * 16, 16)] = dlist[pl.ds(base + jj * 16, 16)]

    def process(base, cp, rbuf):
        stage_idx(base, dstage)
        cp.wait()
        scale_rows(base, rbuf)
        pltpu.sync_copy(rbuf, chunk_sp.at[dstage], add=True)

    def do_flushes(nf):
        def pair(kk, _):
            b0 = (2 * kk) * G
            b1 = b0 + G
            cpA = fire(b0, rowbufA)
            cpB = fire(b1, rowbufB)
            stage_idx(b0, dstage)
            cpA.wait()
            scale_rows(b0, rowbufA)
            scA = pltpu.async_copy(rowbufA, chunk_sp.at[dstage], sem3,
                                   add=True)
            stage_idx(b1, dstageB)
            cpB.wait()
            scale_rows(b1, rowbufB)
            scA.wait()
            pltpu.sync_copy(rowbufB, chunk_sp.at[dstageB], add=True)
            return 0

        lax.fori_loop(0, nf // 2, pair, 0)

        @pl.when(nf % 2 == 1)
        def _():
            b = (nf - 1) * G
            process(b, fire(b, rowbufA), rowbufA)

    def issue_seg(si):
        off = si * 2048
        size = SEGS_M[si]
        par = (si % 2) * 2048
        hoff = tid * SLAB_M + off
        c1 = pltpu.async_copy(dst_hbm.at[pl.ds(hoff, size)],
                              dstb.at[pl.ds(par, size)], sem)
        c2 = pltpu.async_copy(gsrc_hbm.at[pl.ds(hoff, size)],
                              gsb.at[pl.ds(par, size)], sem)
        c3 = pltpu.async_copy(wf_hbm.at[pl.ds(hoff, size)],
                              wb.at[pl.ds(par, size)], sem)
        return (c1, c2, c3)

    def one_pass(p, _):
        base = (2 * p + core) * NCHUNK
        for k in range(TROWS // 112):
            pltpu.sync_copy(zbuf, chunk_sp.at[pl.ds(tid * TROWS + k * 112,
                                                    112)])
        plsc.subcore_barrier()

        cnt = _i32(0)
        cps = issue_seg(0)
        for si in range(NSEG):
            size = SEGS_M[si]
            par = (si % 2) * 2048
            for c in cps:
                c.wait()
            if si + 1 < NSEG:
                cps = issue_seg(si + 1)

            def scan(j, cnt, par=par):
                sl = pl.ds(par + j * 16, 16)
                d16 = dstb[sl]
                m = (d16 >= base) & (d16 < base + NCHUNK)
                plsc.store_compressed(glist.at[pl.ds(cnt, 16)], gsb[sl],
                                      mask=m)
                plsc.store_compressed(dlist.at[pl.ds(cnt, 16)], d16 - base,
                                      mask=m)
                plsc.store_compressed(wlist.at[pl.ds(cnt, 16)], wb[sl],
                                      mask=m)
                return cnt + plsc.all_reduce_population_count(m)[0]

            cnt = lax.fori_loop(0, size // 16, scan, cnt)

            nf = cnt // G
            do_flushes(nf)

            @pl.when(nf > 0)
            def _():
                fb = nf * G
                for jj in range(8):
                    dsl = pl.ds(jj * 16, 16)
                    ssl = pl.ds(fb + jj * 16, 16)
                    glist[dsl] = glist[ssl]
                    dlist[dsl] = dlist[ssl]
                    wlist[dsl] = wlist[ssl]

            cnt = cnt - nf * G

        @pl.when(cnt > 0)
        def _():
            for jj in range(8):
                sl = pl.ds(jj * 16, 16)
                m = (jj * 16 + _iota16()) < cnt
                glist[sl] = jnp.where(m, glist[sl], 0)
                dlist[sl] = jnp.where(m, dlist[sl], 0)
                wlist[sl] = jnp.where(m, wlist[sl], 0.0)
            process(0, fire(0, rowbufA), rowbufA)

        plsc.subcore_barrier()

        if final:
            for k in range(TROWS // 112):
                roff = tid * TROWS + k * 112
                pltpu.sync_copy(chunk_sp.at[pl.ds(roff, 112)],
                                rowbufA.at[pl.ds(0, 112)])

                def badd(i, _):
                    for jj in range(8):
                        sl = pl.ds(jj * 16, 16)
                        rowbufA[i, sl] = rowbufA[i, sl] + bsumv[sl]
                    return 0

                lax.fori_loop(0, 112, badd, 0)
                pltpu.sync_copy(rowbufA.at[pl.ds(0, 112)],
                                agg_hbm.at[pl.ds(base + roff, 112)])
        else:
            pltpu.sync_copy(chunk_sp.at[pl.ds(tid * TROWS, TROWS)],
                            agg_hbm.at[pl.ds(base + tid * TROWS, TROWS)])
        plsc.subcore_barrier()
        return 0

    lax.fori_loop(0, NPASS // 2, one_pass, 0)


def _scatter(final, hw, gsrc, dstf, wf, bias):
    mesh = plsc.VectorSubcoreMesh(core_axis_name="c", subcore_axis_name="s")
    body = functools.partial(_scatter_body, final)
    return pl.kernel(
        body,
        out_type=jax.ShapeDtypeStruct((NPASS * NCHUNK, D), _f32),
        mesh=mesh,
        scratch_types=[
            pltpu.VMEM_SHARED((NCHUNK, D), _f32),
            pltpu.VMEM((4096,), _i32),
            pltpu.VMEM((4096,), _i32),
            pltpu.VMEM((4096,), _f32),
            pltpu.VMEM((LISTCAP,), _i32),
            pltpu.VMEM((LISTCAP,), _i32),
            pltpu.VMEM((LISTCAP,), _f32),
            pltpu.VMEM((G,), _i32),
            pltpu.VMEM((G,), _i32),
            pltpu.VMEM((G, D), _f32),
            pltpu.VMEM((G, D), _f32),
            pltpu.VMEM((112, D), _f32),
            pltpu.VMEM((R, D), _f32),
            pltpu.VMEM((D,), _f32),
            pltpu.SemaphoreType.DMA,
            pltpu.SemaphoreType.DMA,
            pltpu.SemaphoreType.DMA,
        ],
        compiler_params=pltpu.CompilerParams(needs_layout_passes=False),
    )(hw, gsrc, dstf, wf, bias)


def kernel(x, edge_index, Ws, bs):
    ei = edge_index.astype(_i32)
    src3 = jnp.pad(ei[:, 0, :], ((0, 0), (0, PADR - E)))
    dst3 = jnp.pad(ei[:, 1, :], ((0, 0), (0, PADR - E)))
    gsrc = (src3 + jnp.arange(R, dtype=_i32)[:, None] * N).reshape(-1)
    dstf = dst3.reshape(-1)

    idx6 = jnp.pad(jnp.concatenate([ei[:, 0, :], ei[:, 1, :]], axis=0),
                   ((0, 0), (0, PADR - E)))

    s6, _ = _prep(idx6)
    wf = _wbuild(dst3, s6)
    s_out = s6.reshape(6, N_PAD)[0:3, :N].T

    h = x
    for l in range(L):
        hw3 = _mm(l == 0, h, Ws[l], s_out, bs[l - 1] if l > 0 else bs[0])
        hw = hw3.reshape(R * N, D)
        agg = _scatter(l == L - 1, hw, gsrc, dstf, wf, bs[L - 1])
        h = agg[:N]
    return h
